# baseline (speedup 1.0000x reference)
import functools

import jax
import jax.numpy as jnp
from jax import lax
from jax.experimental import pallas as pl
from jax.experimental.pallas import tpu as pltpu
from jax.experimental.pallas import tpu_sc as plsc

NC = 2
NS = 16


def _make_deg(NP, E, CH):
    NW = NC * NS
    TOTC = E // CH
    CPT = TOTC // NW
    TAILC = TOTC - CPT * NW
    RPT = NP // NS
    assert CPT % 2 == 0 and (CH * CPT) % 8 == 0
    mesh = plsc.VectorSubcoreMesh(core_axis_name="c", subcore_axis_name="s")

    @functools.partial(
        pl.kernel,
        out_type=jax.ShapeDtypeStruct((NC * NP,), jnp.float32),
        mesh=mesh,
        scratch_types=[
            pltpu.VMEM((2, CH), jnp.int32),
            pltpu.VMEM((CH,), jnp.float32),
            pltpu.VMEM_SHARED((NP,), jnp.float32),
            pltpu.SemaphoreType.DMA,
            pltpu.SemaphoreType.DMA,
            pltpu.SemaphoreType.DMA,
            pltpu.SemaphoreType.DMA,
        ],
    )
    def deg_kernel(dst_hbm, zeros_hbm, ones_hbm, out_hbm,
                   didx, onesv, deg_sh, semi0, semi1, sems0, sems1):
        semi = (semi0, semi1)
        sems = (sems0, sems1)
        c_ax = lax.axis_index("c")
        s = lax.axis_index("s")
        wid = s * NC + c_ax
        pltpu.sync_copy(zeros_hbm, deg_sh.at[pl.ds(s * RPT, RPT)])
        pltpu.sync_copy(ones_hbm, onesv)
        plsc.subcore_barrier()
        base = wid * (CPT * CH)

        def load_idx(ci, b):
            pltpu.async_copy(dst_hbm.at[pl.ds(base + ci * CH, CH)],
                             didx.at[b], semi[b])

        def wait_idx(b):
            pltpu.make_async_copy(dst_hbm.at[pl.ds(0, CH)],
                                  didx.at[b], semi[b]).wait()

        pltpu.sync_copy(dst_hbm.at[pl.ds(base, CH)], didx.at[0])
        load_idx(1, 1)

        def body(g, carry):
            for b in (0, 1):
                ci = 2 * g + b
                nb = 1 - b
                pltpu.async_copy(onesv, deg_sh.at[didx.at[b]], sems[b],
                                 add=True)

                @pl.when(ci + 1 < CPT)
                def _():
                    wait_idx(nb)

                @pl.when(ci + 2 < CPT)
                def _():
                    pltpu.make_async_copy(onesv, deg_sh.at[pl.ds(0, CH)],
                                          sems[b]).wait()
                    load_idx(ci + 2, b)
            return carry

        lax.fori_loop(0, CPT // 2, body, 0)
        pltpu.make_async_copy(onesv, deg_sh.at[pl.ds(0, CH)], sems0).wait()
        pltpu.make_async_copy(onesv, deg_sh.at[pl.ds(0, CH)], sems1).wait()

        @pl.when(wid < TAILC)
        def _tail():
            off = (CPT * NW + wid) * CH
            pltpu.sync_copy(dst_hbm.at[pl.ds(off, CH)], didx.at[0])
            pltpu.sync_copy(onesv, deg_sh.at[didx.at[0]], add=True)

        plsc.subcore_barrier()
        pltpu.sync_copy(deg_sh.at[pl.ds(s * RPT, RPT)],
                        out_hbm.at[pl.ds(c_ax * NP + s * RPT, RPT)])

    return deg_kernel


def _make_agg(NP, D, E, CH):
    NW = NC * NS
    TOTC = E // CH
    CPT = TOTC // NW
    TAILC = TOTC - CPT * NW
    RPT = NP // NS
    assert (CH * CPT) % 8 == 0 and CPT >= 3
    mesh = plsc.VectorSubcoreMesh(core_axis_name="c", subcore_axis_name="s")

    @functools.partial(
        pl.kernel,
        out_type=jax.ShapeDtypeStruct((NC, NP, D), jnp.float32),
        mesh=mesh,
        scratch_types=[
            pltpu.VMEM((3, CH), jnp.int32),
            pltpu.VMEM((3, CH), jnp.int32),
            pltpu.VMEM((3, CH, D), jnp.float32),
            pltpu.VMEM_SHARED((NP, D), jnp.float32),
            pltpu.SemaphoreType.DMA,
            pltpu.SemaphoreType.DMA,
            pltpu.SemaphoreType.DMA,
            pltpu.SemaphoreType.DMA,
            pltpu.SemaphoreType.DMA,
            pltpu.SemaphoreType.DMA,
        ],
    )
    def agg_kernel(src_hbm, dst_hbm, hs_hbm, zeros_hbm, out_hbm,
                   sidx, didx, rows, agg_sh,
                   semi0, semi1, semi2, semg0, semg1, semg2):
        semi = (semi0, semi1, semi2)
        semg = (semg0, semg1, semg2)
        c_ax = lax.axis_index("c")
        s = lax.axis_index("s")
        wid = s * NC + c_ax
        pltpu.sync_copy(zeros_hbm, agg_sh.at[pl.ds(s * RPT, RPT)])
        plsc.subcore_barrier()
        base = wid * (CPT * CH)

        def load_idx(ci, b):
            off = base + ci * CH
            pltpu.async_copy(src_hbm.at[pl.ds(off, CH)], sidx.at[b], semi[b])
            pltpu.async_copy(dst_hbm.at[pl.ds(off, CH)], didx.at[b], semi[b])

        def wait_idx(b):
            pltpu.make_async_copy(src_hbm.at[pl.ds(0, CH)],
                                  sidx.at[b], semi[b]).wait()
            pltpu.make_async_copy(dst_hbm.at[pl.ds(0, CH)],
                                  didx.at[b], semi[b]).wait()

        def start_gather(b):
            pltpu.async_copy(hs_hbm.at[sidx.at[b]], rows.at[b], semg[b])

        def wait_gather(b):
            pltpu.make_async_copy(hs_hbm.at[pl.ds(0, CH)],
                                  rows.at[b], semg[b]).wait()

        for p in (0, 1):
            pltpu.sync_copy(src_hbm.at[pl.ds(base + p * CH, CH)], sidx.at[p])
            pltpu.sync_copy(dst_hbm.at[pl.ds(base + p * CH, CH)], didx.at[p])
            start_gather(p)
        load_idx(2, 2)

        def body(g, carry):
            for b in (0, 1, 2):
                ci = 3 * g + b
                b2 = (b + 2) % 3

                @pl.when(ci + 2 < CPT)
                def _():
                    wait_idx(b2)
                    start_gather(b2)

                wait_gather(b)
                pltpu.sync_copy(rows.at[b], agg_sh.at[didx.at[b]], add=True)

                @pl.when(ci + 3 < CPT)
                def _():
                    load_idx(ci + 3, b)
            return carry

        lax.fori_loop(0, CPT // 3, body, 0)
        for ci in range(3 * (CPT // 3), CPT):
            b = ci % 3
            wait_gather(b)
            pltpu.sync_copy(rows.at[b], agg_sh.at[didx.at[b]], add=True)

        @pl.when(wid < TAILC)
        def _tail():
            off = (CPT * NW + wid) * CH
            pltpu.sync_copy(src_hbm.at[pl.ds(off, CH)], sidx.at[0])
            pltpu.sync_copy(dst_hbm.at[pl.ds(off, CH)], didx.at[0])
            pltpu.async_copy(hs_hbm.at[sidx.at[0]], rows.at[0], semg0)
            wait_gather(0)
            pltpu.sync_copy(rows.at[0], agg_sh.at[didx.at[0]], add=True)

        plsc.subcore_barrier()
        pltpu.sync_copy(agg_sh.at[pl.ds(s * RPT, RPT)],
                        out_hbm.at[c_ax, pl.ds(s * RPT, RPT)])

    return agg_kernel


def _hs_body(degc_ref, x_ref, w_ref, hs_ref):
    dc = degc_ref[...]
    deg = dc[0] + dc[1] + 1.0
    dinv = lax.rsqrt(deg)
    h = lax.dot_general(x_ref[...], w_ref[...], (((1,), (1,)), ((), ())),
                        preferred_element_type=jnp.float32)
    hs_ref[...] = h * dinv


def _make_hs(N, NP, D, BL):
    return pl.pallas_call(
        _hs_body,
        grid=(NP // BL,),
        in_specs=[
            pl.BlockSpec((NC, BL, 1), lambda i: (0, i, 0)),
            pl.BlockSpec((BL, D), lambda i: (i, 0)),
            pl.BlockSpec((D, D), lambda i: (0, 0)),
        ],
        out_specs=pl.BlockSpec((BL, D), lambda i: (i, 0)),
        out_shape=jax.ShapeDtypeStruct((NP, D), jnp.float32),
    )


def _make_final(N, NP, D, BL):
    nblk = NP // BL

    def body(degc_ref, agg_ref, hs_ref, bc_ref, wl_ref, bl_ref, out_ref, acc):
        i = pl.program_id(0)

        @pl.when(i == 0)
        def _init():
            acc[...] = jnp.zeros_like(acc)

        dc = degc_ref[...]
        deg = dc[0] + dc[1] + 1.0
        dinv = lax.rsqrt(deg)
        a = agg_ref[...]
        row = (a[0] + a[1] + hs_ref[...]) * dinv + bc_ref[...]
        row = jnp.maximum(row, 0.0)
        ridx = lax.broadcasted_iota(jnp.int32, (BL, D), 0) + i * BL
        row = jnp.where(ridx < N, row, 0.0)
        acc[...] += jnp.sum(row, axis=0, keepdims=True)

        @pl.when(i == nblk - 1)
        def _fini():
            v = acc[...] * (1.0 / N)
            z = jnp.sum(v * wl_ref[...], axis=1, keepdims=True) + bl_ref[...]
            score = 1.0 / (1.0 + jnp.exp(-z))
            out_ref[...] = jnp.broadcast_to(score, out_ref.shape)

    return pl.pallas_call(
        body,
        grid=(nblk,),
        in_specs=[
            pl.BlockSpec((NC, BL, 1), lambda i: (0, i, 0)),
            pl.BlockSpec((NC, BL, D), lambda i: (0, i, 0)),
            pl.BlockSpec((BL, D), lambda i: (i, 0)),
            pl.BlockSpec((1, D), lambda i: (0, 0)),
            pl.BlockSpec((1, D), lambda i: (0, 0)),
            pl.BlockSpec((1, 1), lambda i: (0, 0)),
        ],
        out_specs=pl.BlockSpec((8, 128), lambda i: (0, 0)),
        out_shape=jax.ShapeDtypeStruct((8, 128), jnp.float32),
        scratch_shapes=[pltpu.VMEM((1, D), jnp.float32)],
    )


def kernel(x, edge_index, W_conv, b_conv, W_lin, b_lin):
    N, D = x.shape
    E = edge_index.shape[1]
    CHD = 128
    CHA = 80
    BL = 1024

    NP = ((N + BL - 1) // BL) * BL
    ei = edge_index.astype(jnp.int32)
    src = ei[0]
    dst = ei[1]
    zrow = jnp.zeros((NP // NS, D), jnp.float32)
    z1 = jnp.zeros((NP // NS,), jnp.float32)
    ones1 = jnp.ones((CHD,), jnp.float32)

    degf = _make_deg(NP, E, CHD)(dst, z1, ones1)
    degc = degf.reshape(NC, NP, 1)
    hs = _make_hs(N, NP, D, BL)(degc, x, W_conv)
    aggp = _make_agg(NP, D, E, CHA)(src, dst, hs, zrow)
    out = _make_final(N, NP, D, BL)(
        degc, aggp, hs,
        b_conv.reshape(1, D).astype(jnp.float32),
        W_lin.astype(jnp.float32),
        b_lin.reshape(1, 1).astype(jnp.float32),
    )
    return out[0:1, 0:1]


# speedup vs baseline: 1.1456x; 1.1456x over previous
import functools

import jax
import jax.numpy as jnp
from jax import lax
from jax.experimental import pallas as pl
from jax.experimental.pallas import tpu as pltpu
from jax.experimental.pallas import tpu_sc as plsc

NC = 2
NS = 16


def _make_deg(NP, E, CH, RING=8):
    NW = NC * NS
    TOTC = E // CH
    CPT = TOTC // NW
    TAILC = TOTC - CPT * NW
    RPT = NP // NS
    assert TAILC <= NW and (CH * CPT) % 8 == 0
    mesh = plsc.VectorSubcoreMesh(core_axis_name="c", subcore_axis_name="s")

    @functools.partial(
        pl.kernel,
        out_type=jax.ShapeDtypeStruct((NC * NP,), jnp.float32),
        mesh=mesh,
        scratch_types=[
            pltpu.VMEM((CPT + 1, CH), jnp.int32),
            pltpu.VMEM((CH,), jnp.float32),
            pltpu.VMEM_SHARED((NP,), jnp.float32),
            pltpu.SemaphoreType.DMA,
        ],
    )
    def deg_kernel(dst_hbm, dst3_hbm, zeros_hbm, ones_hbm, out_hbm,
                   didx, onesv, deg_sh, sems):
        c_ax = lax.axis_index("c")
        s = lax.axis_index("s")
        wid = s * NC + c_ax
        pltpu.sync_copy(zeros_hbm, deg_sh.at[pl.ds(s * RPT, RPT)])
        pltpu.sync_copy(ones_hbm, onesv)
        pltpu.sync_copy(dst3_hbm.at[wid], didx.at[pl.ds(0, CPT)])

        @pl.when(wid < TAILC)
        def _():
            pltpu.sync_copy(dst_hbm.at[pl.ds((CPT * NW + wid) * CH, CH)],
                            didx.at[CPT])

        plsc.subcore_barrier()
        nch = CPT + jnp.where(wid < TAILC, 1, 0)

        def body(j, carry):
            pltpu.async_copy(onesv, deg_sh.at[didx.at[j]], sems, add=True)

            @pl.when(j >= RING)
            def _():
                pltpu.make_async_copy(onesv, deg_sh.at[pl.ds(0, CH)],
                                      sems).wait()
            return carry

        lax.fori_loop(0, nch, body, 0)

        def drain(j, carry):
            pltpu.make_async_copy(onesv, deg_sh.at[pl.ds(0, CH)], sems).wait()
            return carry

        lax.fori_loop(0, jnp.minimum(nch, RING), drain, 0)
        plsc.subcore_barrier()
        pltpu.sync_copy(deg_sh.at[pl.ds(s * RPT, RPT)],
                        out_hbm.at[pl.ds(c_ax * NP + s * RPT, RPT)])

    return deg_kernel


def _make_agg(NP, D, E, CH):
    NW = NC * NS
    TOTC = E // CH
    CPT = TOTC // NW
    TAILC = TOTC - CPT * NW
    RPT = NP // NS
    EPT = CPT * CH
    assert TAILC == 0 and EPT % 8 == 0 and CPT >= 3
    mesh = plsc.VectorSubcoreMesh(core_axis_name="c", subcore_axis_name="s")

    @functools.partial(
        pl.kernel,
        out_type=jax.ShapeDtypeStruct((NC, NP, D), jnp.float32),
        mesh=mesh,
        scratch_types=[
            pltpu.VMEM((EPT,), jnp.int32),
            pltpu.VMEM((CPT, CH), jnp.int32),
            pltpu.VMEM((2, CH, D), jnp.float32),
            pltpu.VMEM_SHARED((NP, D), jnp.float32),
            pltpu.SemaphoreType.DMA,
            pltpu.SemaphoreType.DMA,
        ],
    )
    def agg_kernel(src_hbm, dst3_hbm, hs_hbm, zeros_hbm, out_hbm,
                   sall, didx, rows, agg_sh, semg0, semg1):
        semg = (semg0, semg1)
        c_ax = lax.axis_index("c")
        s = lax.axis_index("s")
        wid = s * NC + c_ax
        pltpu.sync_copy(zeros_hbm, agg_sh.at[pl.ds(s * RPT, RPT)])
        pltpu.sync_copy(src_hbm.at[pl.ds(wid * EPT, EPT)], sall)
        pltpu.sync_copy(dst3_hbm.at[wid], didx)
        plsc.subcore_barrier()

        def start_gather(ci, b):
            pltpu.async_copy(hs_hbm.at[sall.at[pl.ds(ci * CH, CH)]],
                             rows.at[b], semg[b])

        def wait_gather(b):
            pltpu.make_async_copy(hs_hbm.at[pl.ds(0, CH)],
                                  rows.at[b], semg[b]).wait()

        start_gather(0, 0)
        start_gather(1, 1)

        def body(g, carry):
            for b in (0, 1):
                ci = 2 * g + b
                wait_gather(b)
                pltpu.sync_copy(rows.at[b], agg_sh.at[didx.at[ci]], add=True)

                @pl.when(ci + 2 < CPT)
                def _():
                    start_gather(ci + 2, b)
            return carry

        lax.fori_loop(0, CPT // 2, body, 0)
        for ci in range(2 * (CPT // 2), CPT):
            b = ci % 2
            wait_gather(b)
            pltpu.sync_copy(rows.at[b], agg_sh.at[didx.at[ci]], add=True)

        plsc.subcore_barrier()
        pltpu.sync_copy(agg_sh.at[pl.ds(s * RPT, RPT)],
                        out_hbm.at[c_ax, pl.ds(s * RPT, RPT)])

    return agg_kernel


def _hs_body(degc_ref, x_ref, w_ref, hs_ref):
    dc = degc_ref[...]
    deg = dc[0] + dc[1] + 1.0
    dinv = lax.rsqrt(deg)
    h = lax.dot_general(x_ref[...], w_ref[...], (((1,), (1,)), ((), ())),
                        preferred_element_type=jnp.float32)
    hs_ref[...] = h * dinv


def _make_hs(N, NP, D, BL):
    return pl.pallas_call(
        _hs_body,
        grid=(NP // BL,),
        in_specs=[
            pl.BlockSpec((NC, BL, 1), lambda i: (0, i, 0)),
            pl.BlockSpec((BL, D), lambda i: (i, 0)),
            pl.BlockSpec((D, D), lambda i: (0, 0)),
        ],
        out_specs=pl.BlockSpec((BL, D), lambda i: (i, 0)),
        out_shape=jax.ShapeDtypeStruct((NP, D), jnp.float32),
    )


def _make_final(N, NP, D, BL):
    nblk = NP // BL

    def body(degc_ref, agg_ref, hs_ref, bc_ref, wl_ref, bl_ref, out_ref, acc):
        i = pl.program_id(0)

        @pl.when(i == 0)
        def _init():
            acc[...] = jnp.zeros_like(acc)

        dc = degc_ref[...]
        deg = dc[0] + dc[1] + 1.0
        dinv = lax.rsqrt(deg)
        a = agg_ref[...]
        row = (a[0] + a[1] + hs_ref[...]) * dinv + bc_ref[...]
        row = jnp.maximum(row, 0.0)
        ridx = lax.broadcasted_iota(jnp.int32, (BL, D), 0) + i * BL
        row = jnp.where(ridx < N, row, 0.0)
        acc[...] += jnp.sum(row, axis=0, keepdims=True)

        @pl.when(i == nblk - 1)
        def _fini():
            v = acc[...] * (1.0 / N)
            z = jnp.sum(v * wl_ref[...], axis=1, keepdims=True) + bl_ref[...]
            score = 1.0 / (1.0 + jnp.exp(-z))
            out_ref[...] = jnp.broadcast_to(score, out_ref.shape)

    return pl.pallas_call(
        body,
        grid=(nblk,),
        in_specs=[
            pl.BlockSpec((NC, BL, 1), lambda i: (0, i, 0)),
            pl.BlockSpec((NC, BL, D), lambda i: (0, i, 0)),
            pl.BlockSpec((BL, D), lambda i: (i, 0)),
            pl.BlockSpec((1, D), lambda i: (0, 0)),
            pl.BlockSpec((1, D), lambda i: (0, 0)),
            pl.BlockSpec((1, 1), lambda i: (0, 0)),
        ],
        out_specs=pl.BlockSpec((8, 128), lambda i: (0, 0)),
        out_shape=jax.ShapeDtypeStruct((8, 128), jnp.float32),
        scratch_shapes=[pltpu.VMEM((1, D), jnp.float32)],
    )


def kernel(x, edge_index, W_conv, b_conv, W_lin, b_lin):
    N, D = x.shape
    E = edge_index.shape[1]
    CHD = 128
    CHA = 80
    BL = 1024

    NP = ((N + BL - 1) // BL) * BL
    ei = edge_index.astype(jnp.int32)
    src = ei[0]
    dst = ei[1]
    zrow = jnp.zeros((NP // NS, D), jnp.float32)
    z1 = jnp.zeros((NP // NS,), jnp.float32)
    ones1 = jnp.ones((CHD,), jnp.float32)

    NW = NC * NS
    CPTD = (E // CHD) // NW
    dst3d = dst[:CPTD * NW * CHD].reshape(NW, CPTD, CHD)
    dst3a = dst.reshape(NW, (E // NW) // CHA, CHA)

    degf = _make_deg(NP, E, CHD)(dst, dst3d, z1, ones1)
    degc = degf.reshape(NC, NP, 1)
    hs = _make_hs(N, NP, D, BL)(degc, x, W_conv)
    aggp = _make_agg(NP, D, E, CHA)(src, dst3a, hs, zrow)
    out = _make_final(N, NP, D, BL)(
        degc, aggp, hs,
        b_conv.reshape(1, D).astype(jnp.float32),
        W_lin.astype(jnp.float32),
        b_lin.reshape(1, 1).astype(jnp.float32),
    )
    return out[0:1, 0:1]


# speedup vs baseline: 1.2253x; 1.0696x over previous
import functools

import jax
import jax.numpy as jnp
from jax import lax
from jax.experimental import pallas as pl
from jax.experimental.pallas import tpu as pltpu
from jax.experimental.pallas import tpu_sc as plsc

NC = 2
NS = 16


def _make_deg(NP, E, CH, RING=8):
    NW = NC * NS
    TOTC = E // CH
    CPT = TOTC // NW
    TAILC = TOTC - CPT * NW
    RPT = NP // NS
    assert TAILC <= NW and (CH * CPT) % 8 == 0
    mesh = plsc.VectorSubcoreMesh(core_axis_name="c", subcore_axis_name="s")

    @functools.partial(
        pl.kernel,
        out_type=jax.ShapeDtypeStruct((NC * NP,), jnp.float32),
        mesh=mesh,
        scratch_types=[
            pltpu.VMEM((CPT + 1, CH), jnp.int32),
            pltpu.VMEM((CH,), jnp.float32),
            pltpu.VMEM_SHARED((NP,), jnp.float32),
            pltpu.SemaphoreType.DMA,
        ],
    )
    def deg_kernel(dst_hbm, dst3_hbm, zeros_hbm, ones_hbm, out_hbm,
                   didx, onesv, deg_sh, sems):
        c_ax = lax.axis_index("c")
        s = lax.axis_index("s")
        wid = s * NC + c_ax
        pltpu.sync_copy(zeros_hbm, deg_sh.at[pl.ds(s * RPT, RPT)])
        pltpu.sync_copy(ones_hbm, onesv)
        pltpu.sync_copy(dst3_hbm.at[wid], didx.at[pl.ds(0, CPT)])

        @pl.when(wid < TAILC)
        def _():
            pltpu.sync_copy(dst_hbm.at[pl.ds((CPT * NW + wid) * CH, CH)],
                            didx.at[CPT])

        plsc.subcore_barrier()
        nch = CPT + jnp.where(wid < TAILC, 1, 0)

        def body(j, carry):
            pltpu.async_copy(onesv, deg_sh.at[didx.at[j]], sems, add=True)

            @pl.when(j >= RING)
            def _():
                pltpu.make_async_copy(onesv, deg_sh.at[pl.ds(0, CH)],
                                      sems).wait()
            return carry

        lax.fori_loop(0, nch, body, 0)

        def drain(j, carry):
            pltpu.make_async_copy(onesv, deg_sh.at[pl.ds(0, CH)], sems).wait()
            return carry

        lax.fori_loop(0, jnp.minimum(nch, RING), drain, 0)
        plsc.subcore_barrier()
        pltpu.sync_copy(deg_sh.at[pl.ds(s * RPT, RPT)],
                        out_hbm.at[pl.ds(c_ax * NP + s * RPT, RPT)])

    return deg_kernel


def _make_agg(NP, D, E, CH):
    NW = NC * NS
    TOTC = E // CH
    CPT = TOTC // NW
    TAILC = TOTC - CPT * NW
    RPT = NP // NS
    EPT = CPT * CH
    assert TAILC <= NW and EPT % 8 == 0 and CPT % 6 == 0
    mesh = plsc.VectorSubcoreMesh(core_axis_name="c", subcore_axis_name="s")

    @functools.partial(
        pl.kernel,
        out_type=jax.ShapeDtypeStruct((NC, NP, D), jnp.float32),
        mesh=mesh,
        scratch_types=[
            pltpu.VMEM((3, CH), jnp.int32),
            pltpu.VMEM((CPT + 1, CH), jnp.int32),
            pltpu.VMEM((2, CH, D), jnp.float32),
            pltpu.VMEM_SHARED((NP, D), jnp.float32),
            pltpu.SemaphoreType.DMA,
            pltpu.SemaphoreType.DMA,
            pltpu.SemaphoreType.DMA,
            pltpu.SemaphoreType.DMA,
            pltpu.SemaphoreType.DMA,
        ],
    )
    def agg_kernel(src_hbm, dst_hbm, dst3_hbm, hs_hbm, zeros_hbm, out_hbm,
                   sidx, didx, rows, agg_sh,
                   semi0, semi1, semi2, semg0, semg1):
        semi = (semi0, semi1, semi2)
        semg = (semg0, semg1)
        c_ax = lax.axis_index("c")
        s = lax.axis_index("s")
        wid = s * NC + c_ax

        @pl.when(c_ax == 0)
        def _():
            pltpu.sync_copy(hs_hbm.at[pl.ds(s * RPT, RPT)],
                            agg_sh.at[pl.ds(s * RPT, RPT)])

        @pl.when(c_ax != 0)
        def _():
            pltpu.sync_copy(zeros_hbm, agg_sh.at[pl.ds(s * RPT, RPT)])

        pltpu.sync_copy(dst3_hbm.at[wid], didx.at[pl.ds(0, CPT)])

        @pl.when(wid < TAILC)
        def _():
            pltpu.sync_copy(dst_hbm.at[pl.ds((CPT * NW + wid) * CH, CH)],
                            didx.at[CPT])

        base = wid * EPT

        def load_idx(ci, r):
            pltpu.async_copy(src_hbm.at[pl.ds(base + ci * CH, CH)],
                             sidx.at[r], semi[r])

        def wait_idx(r):
            pltpu.make_async_copy(src_hbm.at[pl.ds(0, CH)],
                                  sidx.at[r], semi[r]).wait()

        def start_gather(r, b):
            pltpu.async_copy(hs_hbm.at[sidx.at[r]], rows.at[b], semg[b])

        def wait_gather(b):
            pltpu.make_async_copy(hs_hbm.at[pl.ds(0, CH)],
                                  rows.at[b], semg[b]).wait()

        plsc.subcore_barrier()
        pltpu.sync_copy(src_hbm.at[pl.ds(base, CH)], sidx.at[0])
        pltpu.sync_copy(src_hbm.at[pl.ds(base + CH, CH)], sidx.at[1])
        start_gather(0, 0)
        start_gather(1, 1)
        load_idx(2, 2)

        def body(g, carry):
            for u in range(6):
                ci = 6 * g + u
                b = u % 2
                r = u % 3
                wait_gather(b)
                pltpu.sync_copy(rows.at[b], agg_sh.at[didx.at[ci]], add=True)

                @pl.when(ci + 2 < CPT)
                def _():
                    wait_idx((u + 2) % 3)
                    start_gather((u + 2) % 3, b)

                @pl.when(ci + 3 < CPT)
                def _():
                    load_idx(ci + 3, r)
            return carry

        lax.fori_loop(0, CPT // 6, body, 0)

        @pl.when(wid < TAILC)
        def _tail():
            off = (CPT * NW + wid) * CH
            pltpu.sync_copy(src_hbm.at[pl.ds(off, CH)], sidx.at[0])
            start_gather(0, 0)
            wait_gather(0)
            pltpu.sync_copy(rows.at[0], agg_sh.at[didx.at[CPT]], add=True)

        plsc.subcore_barrier()
        pltpu.sync_copy(agg_sh.at[pl.ds(s * RPT, RPT)],
                        out_hbm.at[c_ax, pl.ds(s * RPT, RPT)])

    return agg_kernel


def _hs_body(degc_ref, x_ref, w_ref, hs_ref):
    dc = degc_ref[...]
    deg = dc[0] + dc[1] + 1.0
    dinv = lax.rsqrt(deg)
    h = lax.dot_general(x_ref[...], w_ref[...], (((1,), (1,)), ((), ())),
                        preferred_element_type=jnp.float32)
    hs_ref[...] = h * dinv


def _make_hs(N, NP, D, BL):
    return pl.pallas_call(
        _hs_body,
        grid=(NP // BL,),
        in_specs=[
            pl.BlockSpec((NC, BL, 1), lambda i: (0, i, 0)),
            pl.BlockSpec((BL, D), lambda i: (i, 0)),
            pl.BlockSpec((D, D), lambda i: (0, 0)),
        ],
        out_specs=pl.BlockSpec((BL, D), lambda i: (i, 0)),
        out_shape=jax.ShapeDtypeStruct((NP, D), jnp.float32),
    )


def _make_final(N, NP, D, BL):
    nblk = NP // BL

    def body(degc_ref, agg_ref, bc_ref, wl_ref, bl_ref, out_ref, acc):
        i = pl.program_id(0)

        @pl.when(i == 0)
        def _init():
            acc[...] = jnp.zeros_like(acc)

        dc = degc_ref[...]
        deg = dc[0] + dc[1] + 1.0
        dinv = lax.rsqrt(deg)
        a = agg_ref[...]
        row = (a[0] + a[1]) * dinv + bc_ref[...]
        row = jnp.maximum(row, 0.0)
        ridx = lax.broadcasted_iota(jnp.int32, (BL, D), 0) + i * BL
        row = jnp.where(ridx < N, row, 0.0)
        acc[...] += jnp.sum(row, axis=0, keepdims=True)

        @pl.when(i == nblk - 1)
        def _fini():
            v = acc[...] * (1.0 / N)
            z = jnp.sum(v * wl_ref[...], axis=1, keepdims=True) + bl_ref[...]
            score = 1.0 / (1.0 + jnp.exp(-z))
            out_ref[...] = jnp.broadcast_to(score, out_ref.shape)

    return pl.pallas_call(
        body,
        grid=(nblk,),
        in_specs=[
            pl.BlockSpec((NC, BL, 1), lambda i: (0, i, 0)),
            pl.BlockSpec((NC, BL, D), lambda i: (0, i, 0)),
            pl.BlockSpec((1, D), lambda i: (0, 0)),
            pl.BlockSpec((1, D), lambda i: (0, 0)),
            pl.BlockSpec((1, 1), lambda i: (0, 0)),
        ],
        out_specs=pl.BlockSpec((8, 128), lambda i: (0, 0)),
        out_shape=jax.ShapeDtypeStruct((8, 128), jnp.float32),
        scratch_shapes=[pltpu.VMEM((1, D), jnp.float32)],
    )


def kernel(x, edge_index, W_conv, b_conv, W_lin, b_lin):
    N, D = x.shape
    E = edge_index.shape[1]
    CH = 128
    BL = 1024

    NP = ((N + BL - 1) // BL) * BL
    ei = edge_index.astype(jnp.int32)
    src = ei[0]
    dst = ei[1]
    zrow = jnp.zeros((NP // NS, D), jnp.float32)
    z1 = jnp.zeros((NP // NS,), jnp.float32)
    ones1 = jnp.ones((CH,), jnp.float32)

    NW = NC * NS
    CPT = (E // CH) // NW
    dst3 = dst[:CPT * NW * CH].reshape(NW, CPT, CH)

    degf = _make_deg(NP, E, CH)(dst, dst3, z1, ones1)
    degc = degf.reshape(NC, NP, 1)
    hs = _make_hs(N, NP, D, BL)(degc, x, W_conv)
    aggp = _make_agg(NP, D, E, CH)(src, dst, dst3, hs, zrow)
    out = _make_final(N, NP, D, BL)(
        degc, aggp,
        b_conv.reshape(1, D).astype(jnp.float32),
        W_lin.astype(jnp.float32),
        b_lin.reshape(1, 1).astype(jnp.float32),
    )
    return out[0:1, 0:1]


# speedup vs baseline: 1.3315x; 1.0867x over previous
import functools

import jax
import jax.numpy as jnp
from jax import lax
from jax.experimental import pallas as pl
from jax.experimental.pallas import tpu as pltpu
from jax.experimental.pallas import tpu_sc as plsc

NC = 2
NS = 16


def _make_deg(NP, E, CH, RING=16):
    NW = NC * NS
    TOTC = E // CH
    CPT = TOTC // NW
    TAILC = TOTC - CPT * NW
    RPT = NP // NS
    assert TAILC <= NW and (CH * CPT) % 8 == 0
    mesh = plsc.VectorSubcoreMesh(core_axis_name="c", subcore_axis_name="s")

    @functools.partial(
        pl.kernel,
        out_type=jax.ShapeDtypeStruct((NC * NP,), jnp.float32),
        mesh=mesh,
        scratch_types=[
            pltpu.VMEM((CPT + 1, CH), jnp.int32),
            pltpu.VMEM((CH,), jnp.float32),
            pltpu.VMEM_SHARED((NP,), jnp.float32),
            pltpu.SemaphoreType.DMA,
        ],
    )
    def deg_kernel(ei_hbm, dst3_hbm, zeros_hbm, ones_hbm, out_hbm,
                   didx, onesv, deg_sh, sems):
        c_ax = lax.axis_index("c")
        s = lax.axis_index("s")
        wid = s * NC + c_ax
        pltpu.sync_copy(zeros_hbm, deg_sh.at[pl.ds(s * RPT, RPT)])
        pltpu.sync_copy(ones_hbm, onesv)
        pltpu.sync_copy(dst3_hbm.at[wid], didx.at[pl.ds(0, CPT)])

        @pl.when(wid < TAILC)
        def _():
            pltpu.sync_copy(ei_hbm.at[1, pl.ds((CPT * NW + wid) * CH, CH)],
                            didx.at[CPT])

        plsc.subcore_barrier()
        nch = CPT + jnp.where(wid < TAILC, 1, 0)

        def body(j, carry):
            pltpu.async_copy(onesv, deg_sh.at[didx.at[j]], sems, add=True)

            @pl.when(j >= RING)
            def _():
                pltpu.make_async_copy(onesv, deg_sh.at[pl.ds(0, CH)],
                                      sems).wait()
            return carry

        lax.fori_loop(0, nch, body, 0)

        def drain(j, carry):
            pltpu.make_async_copy(onesv, deg_sh.at[pl.ds(0, CH)], sems).wait()
            return carry

        lax.fori_loop(0, jnp.minimum(nch, RING), drain, 0)
        plsc.subcore_barrier()
        pltpu.sync_copy(deg_sh.at[pl.ds(s * RPT, RPT)],
                        out_hbm.at[pl.ds(c_ax * NP + s * RPT, RPT)])

    return deg_kernel


def _make_agg(NP, D, E, CH):
    NW = NC * NS
    TOTC = E // CH
    CPT = TOTC // NW
    TAILC = TOTC - CPT * NW
    RPT = NP // NS
    EPT = CPT * CH
    assert TAILC <= NW and EPT % 8 == 0 and CPT % 6 == 0
    mesh = plsc.VectorSubcoreMesh(core_axis_name="c", subcore_axis_name="s")

    @functools.partial(
        pl.kernel,
        out_type=jax.ShapeDtypeStruct((NC, NP, D), jnp.float32),
        mesh=mesh,
        scratch_types=[
            pltpu.VMEM((3, CH), jnp.int32),
            pltpu.VMEM((CPT + 1, CH), jnp.int32),
            pltpu.VMEM((2, CH, D), jnp.float32),
            pltpu.VMEM_SHARED((NP, D), jnp.float32),
            pltpu.SemaphoreType.DMA,
            pltpu.SemaphoreType.DMA,
            pltpu.SemaphoreType.DMA,
            pltpu.SemaphoreType.DMA,
            pltpu.SemaphoreType.DMA,
        ],
    )
    def agg_kernel(ei_hbm, dst3_hbm, hs_hbm, zeros_hbm, out_hbm,
                   sidx, didx, rows, agg_sh,
                   semi0, semi1, semi2, semg0, semg1):
        semi = (semi0, semi1, semi2)
        semg = (semg0, semg1)
        c_ax = lax.axis_index("c")
        s = lax.axis_index("s")
        wid = s * NC + c_ax

        @pl.when(c_ax == 0)
        def _():
            pltpu.sync_copy(hs_hbm.at[pl.ds(s * RPT, RPT)],
                            agg_sh.at[pl.ds(s * RPT, RPT)])

        @pl.when(c_ax != 0)
        def _():
            pltpu.sync_copy(zeros_hbm, agg_sh.at[pl.ds(s * RPT, RPT)])

        pltpu.sync_copy(dst3_hbm.at[wid], didx.at[pl.ds(0, CPT)])

        @pl.when(wid < TAILC)
        def _():
            pltpu.sync_copy(ei_hbm.at[1, pl.ds((CPT * NW + wid) * CH, CH)],
                            didx.at[CPT])

        base = wid * EPT

        def load_idx(ci, r):
            pltpu.async_copy(ei_hbm.at[0, pl.ds(base + ci * CH, CH)],
                             sidx.at[r], semi[r])

        def wait_idx(r):
            pltpu.make_async_copy(ei_hbm.at[0, pl.ds(0, CH)],
                                  sidx.at[r], semi[r]).wait()

        def start_gather(r, b):
            pltpu.async_copy(hs_hbm.at[sidx.at[r]], rows.at[b], semg[b])

        def wait_gather(b):
            pltpu.make_async_copy(hs_hbm.at[pl.ds(0, CH)],
                                  rows.at[b], semg[b]).wait()

        plsc.subcore_barrier()
        pltpu.sync_copy(ei_hbm.at[0, pl.ds(base, CH)], sidx.at[0])
        pltpu.sync_copy(ei_hbm.at[0, pl.ds(base + CH, CH)], sidx.at[1])
        start_gather(0, 0)
        start_gather(1, 1)
        load_idx(2, 2)

        def body(g, carry):
            for u in range(6):
                ci = 6 * g + u
                b = u % 2
                r = u % 3
                wait_gather(b)
                pltpu.sync_copy(rows.at[b], agg_sh.at[didx.at[ci]], add=True)

                @pl.when(ci + 2 < CPT)
                def _():
                    wait_idx((u + 2) % 3)
                    start_gather((u + 2) % 3, b)

                @pl.when(ci + 3 < CPT)
                def _():
                    load_idx(ci + 3, r)
            return carry

        lax.fori_loop(0, CPT // 6, body, 0)

        @pl.when(wid < TAILC)
        def _tail():
            off = (CPT * NW + wid) * CH
            pltpu.sync_copy(ei_hbm.at[0, pl.ds(off, CH)], sidx.at[0])
            start_gather(0, 0)
            wait_gather(0)
            pltpu.sync_copy(rows.at[0], agg_sh.at[didx.at[CPT]], add=True)

        plsc.subcore_barrier()
        pltpu.sync_copy(agg_sh.at[pl.ds(s * RPT, RPT)],
                        out_hbm.at[c_ax, pl.ds(s * RPT, RPT)])

    return agg_kernel


def _hs_body(degc_ref, x_ref, w_ref, hs_ref):
    dc = degc_ref[...]
    deg = dc[0] + dc[1] + 1.0
    dinv = lax.rsqrt(deg)
    h = lax.dot_general(x_ref[...], w_ref[...], (((1,), (1,)), ((), ())),
                        preferred_element_type=jnp.float32)
    hs_ref[...] = h * dinv


def _make_hs(N, NP, D, BL):
    return pl.pallas_call(
        _hs_body,
        grid=(NP // BL,),
        in_specs=[
            pl.BlockSpec((NC, BL, 1), lambda i: (0, i, 0)),
            pl.BlockSpec((BL, D), lambda i: (i, 0)),
            pl.BlockSpec((D, D), lambda i: (0, 0)),
        ],
        out_specs=pl.BlockSpec((BL, D), lambda i: (i, 0)),
        out_shape=jax.ShapeDtypeStruct((NP, D), jnp.float32),
    )


def _make_final(N, NP, D, BL):
    nblk = NP // BL

    def body(degc_ref, agg_ref, bc_ref, wl_ref, bl_ref, out_ref, acc):
        i = pl.program_id(0)

        @pl.when(i == 0)
        def _init():
            acc[...] = jnp.zeros_like(acc)

        dc = degc_ref[...]
        deg = dc[0] + dc[1] + 1.0
        dinv = lax.rsqrt(deg)
        a = agg_ref[...]
        row = (a[0] + a[1]) * dinv + bc_ref[...]
        row = jnp.maximum(row, 0.0)
        ridx = lax.broadcasted_iota(jnp.int32, (BL, D), 0) + i * BL
        row = jnp.where(ridx < N, row, 0.0)
        acc[...] += jnp.sum(row, axis=0, keepdims=True)

        @pl.when(i == nblk - 1)
        def _fini():
            v = acc[...] * (1.0 / N)
            z = jnp.sum(v * wl_ref[...], axis=1, keepdims=True) + bl_ref[...]
            score = 1.0 / (1.0 + jnp.exp(-z))
            out_ref[...] = jnp.broadcast_to(score, out_ref.shape)

    return pl.pallas_call(
        body,
        grid=(nblk,),
        in_specs=[
            pl.BlockSpec((NC, BL, 1), lambda i: (0, i, 0)),
            pl.BlockSpec((NC, BL, D), lambda i: (0, i, 0)),
            pl.BlockSpec((1, D), lambda i: (0, 0)),
            pl.BlockSpec((1, D), lambda i: (0, 0)),
            pl.BlockSpec((1, 1), lambda i: (0, 0)),
        ],
        out_specs=pl.BlockSpec((8, 128), lambda i: (0, 0)),
        out_shape=jax.ShapeDtypeStruct((8, 128), jnp.float32),
        scratch_shapes=[pltpu.VMEM((1, D), jnp.float32)],
    )


def kernel(x, edge_index, W_conv, b_conv, W_lin, b_lin):
    N, D = x.shape
    E = edge_index.shape[1]
    CH = 128
    BL = 1024

    NP = ((N + BL - 1) // BL) * BL
    ei = edge_index.astype(jnp.int32)
    zrow = jnp.zeros((NP // NS, D), jnp.float32)
    z1 = jnp.zeros((NP // NS,), jnp.float32)
    ones1 = jnp.ones((CH,), jnp.float32)

    NW = NC * NS
    CPT = (E // CH) // NW
    dst3 = ei[1, :CPT * NW * CH].reshape(NW, CPT, CH)

    degf = _make_deg(NP, E, CH)(ei, dst3, z1, ones1)
    degc = degf.reshape(NC, NP, 1)
    hs = _make_hs(N, NP, D, BL)(degc, x, W_conv)
    aggp = _make_agg(NP, D, E, CH)(ei, dst3, hs, zrow)
    out = _make_final(N, NP, D, BL)(
        degc, aggp,
        b_conv.reshape(1, D).astype(jnp.float32),
        W_lin.astype(jnp.float32),
        b_lin.reshape(1, 1).astype(jnp.float32),
    )
    return out[0:1, 0:1]
